# baseline (speedup 1.0000x reference)
import jax
import jax.numpy as jnp
from jax import lax
from jax.experimental import pallas as pl
from jax.experimental.pallas import tpu as pltpu
from jax.experimental.pallas import tpu_sc as plsc

_FIELD0 = 1000000
_B = 16384
_D = 16
_NC, _NS = 2, 16
_NW = _NC * _NS
_BPW = _B // _NW
_NG = _BPW // _D
_EPW = _BPW * 2 * _D
_CH = 128
_GE = 2 * _D * _D
_NQ = 4
_GPQ = _NG // _NQ
_QE = _GPQ * _GE
_SLAB = 8 * _FIELD0 * 2


def _body(xf_hbm, tflat_hbm, out_hbm, xi0_v, xi1_v, idx_v, dat_v, out_v,
          s0, s1, s2, s3):
  sems = (s0, s1, s2, s3)
  wid = lax.axis_index("s") * _NC + lax.axis_index("c")
  base = wid * _BPW

  for k in range(_BPW // _CH):
    xb = (wid * 4 + k) * 2 * _CH
    pltpu.sync_copy(xf_hbm.at[pl.ds(xb, _CH)], xi0_v.at[pl.ds(k * _CH, _CH)])
    pltpu.sync_copy(xf_hbm.at[pl.ds(xb + _CH, _CH)],
                    xi1_v.at[pl.ds(k * _CH, _CH)])

  def build_and_fire(g, sem):
    row = pl.ds(g * _D, _D)
    for f, ref, off in ((0, xi0_v, 0), (1, xi1_v, _FIELD0)):
      iv = ref[row] + off
      ebase = (lax.shift_right_logical(iv, 7) << 10) + (iv & 127)
      for j in range(_D):
        s = f * _D + j
        jo = (j >> 3) * _SLAB + (j & 7) * _CH
        idx_v[pl.ds((g * 2 * _D + s) * _D, _D)] = ebase + jo
    for c in range(4):
      sl = pl.ds(g * _GE + c * _CH, _CH)
      pltpu.async_copy(tflat_hbm.at[idx_v.at[sl]], dat_v.at[sl], sem)

  def dot(g, carry):
    eb = g * _GE
    acc = jnp.zeros((_D,), jnp.float32)
    for j in range(_D):
      a = dat_v[pl.ds(eb + j * _D, _D)]
      b = dat_v[pl.ds(eb + (_D + j) * _D, _D)]
      acc = acc + a * b
    e = jnp.exp(-jnp.abs(acc))
    out_v[pl.ds(g * _D, _D)] = jnp.where(acc >= 0.0, 1.0 / (1.0 + e),
                                         e / (1.0 + e))
    return carry

  for q in range(_NQ):
    def bf(g, carry, _sem=sems[q], _q=q):
      build_and_fire(_q * _GPQ + g, _sem)
      return carry
    lax.fori_loop(0, _GPQ, bf, 0)
  for q in range(_NQ):
    pltpu.make_async_copy(tflat_hbm.at[pl.ds(0, _QE)],
                          dat_v.at[pl.ds(q * _QE, _QE)], sems[q]).wait()
    lax.fori_loop(q * _GPQ, (q + 1) * _GPQ, dot, 0)

  pltpu.sync_copy(out_v, out_hbm.at[pl.ds(base, _BPW)])


@jax.jit
def _run(xf, tflat):
  mesh = plsc.VectorSubcoreMesh(core_axis_name="c", subcore_axis_name="s",
                                num_cores=_NC, num_subcores=_NS)
  return pl.kernel(
      _body,
      out_type=jax.ShapeDtypeStruct((_B,), jnp.float32),
      mesh=mesh,
      compiler_params=pltpu.CompilerParams(needs_layout_passes=False,
                                           use_tc_tiling_on_sc=False),
      scratch_types=[
          pltpu.VMEM((_BPW,), jnp.int32),
          pltpu.VMEM((_BPW,), jnp.int32),
          pltpu.VMEM((_EPW,), jnp.int32),
          pltpu.VMEM((_EPW,), jnp.float32),
          pltpu.VMEM((_BPW,), jnp.float32),
          pltpu.SemaphoreType.DMA,
          pltpu.SemaphoreType.DMA,
          pltpu.SemaphoreType.DMA,
          pltpu.SemaphoreType.DMA,
      ],
  )(xf, tflat)


def kernel(x, table):
  n, b = table.shape[0], x.shape[0]
  xf = (jnp.asarray(x, jnp.int32).T.reshape(2, b // 128, 128)
        .transpose(1, 0, 2).reshape(2 * b))
  tflat = (table.T.reshape(2, 8, n // 128, 128)
           .transpose(0, 2, 1, 3).reshape(n * _D))
  return _run(xf, tflat).reshape(b, 1)


# speedup vs baseline: 1.0751x; 1.0751x over previous
import jax
import jax.numpy as jnp
from jax import lax
from jax.experimental import pallas as pl
from jax.experimental.pallas import tpu as pltpu
from jax.experimental.pallas import tpu_sc as plsc

_FIELD0 = 1000000
_B = 16384
_D = 16
_NC, _NS = 2, 16
_NW = _NC * _NS
_BPW = _B // _NW
_NG = _BPW // _D
_EPW = _BPW * 2 * _D
_CH = 128
_GE = 2 * _D * _D
_NQ = 4
_GPQ = _NG // _NQ
_QE = _GPQ * _GE
_SLAB = 8 * _FIELD0 * 2


def _body(xf_hbm, tflat_hbm, out_hbm, xi0_v, xi1_v, idx_v, dat_v, out_v,
          s0, s1, s2, s3):
  sems = (s0, s1, s2, s3)
  wid = lax.axis_index("s") * _NC + lax.axis_index("c")
  base = wid * _BPW

  for k in range(_BPW // _CH):
    xb = (wid * 4 + k) * 2 * _CH
    pltpu.async_copy(xf_hbm.at[pl.ds(xb, _CH)],
                     xi0_v.at[pl.ds(k * _CH, _CH)], s0)
    pltpu.async_copy(xf_hbm.at[pl.ds(xb + _CH, _CH)],
                     xi1_v.at[pl.ds(k * _CH, _CH)], s0)
  pltpu.make_async_copy(xf_hbm.at[pl.ds(0, _BPW)], xi0_v, s0).wait()
  pltpu.make_async_copy(xf_hbm.at[pl.ds(0, _BPW)], xi1_v, s0).wait()

  def build_and_fire(g, sem):
    row = pl.ds(g * _D, _D)
    for f, ref, off in ((0, xi0_v, 0), (1, xi1_v, _FIELD0)):
      iv = ref[row] + off
      ebase = (lax.shift_right_logical(iv, 7) << 10) + (iv & 127)
      for j in range(_D):
        s = f * _D + j
        jo = (j >> 3) * _SLAB + (j & 7) * _CH
        idx_v[pl.ds((g * 2 * _D + s) * _D, _D)] = ebase + jo
    for c in range(4):
      sl = pl.ds(g * _GE + c * _CH, _CH)
      pltpu.async_copy(tflat_hbm.at[idx_v.at[sl]], dat_v.at[sl], sem)

  def dot(g, carry):
    eb = g * _GE
    acc = jnp.zeros((_D,), jnp.float32)
    for j in range(_D):
      a = dat_v[pl.ds(eb + j * _D, _D)]
      b = dat_v[pl.ds(eb + (_D + j) * _D, _D)]
      acc = acc + a * b
    e = jnp.exp(-jnp.abs(acc))
    out_v[pl.ds(g * _D, _D)] = jnp.where(acc >= 0.0, 1.0 / (1.0 + e),
                                         e / (1.0 + e))
    return carry

  for q in range(_NQ):
    def bf(g, carry, _sem=sems[q], _q=q):
      build_and_fire(_q * _GPQ + g, _sem)
      return carry
    lax.fori_loop(0, _GPQ, bf, 0)
  for q in range(_NQ):
    pltpu.make_async_copy(tflat_hbm.at[pl.ds(0, _QE)],
                          dat_v.at[pl.ds(q * _QE, _QE)], sems[q]).wait()
    lax.fori_loop(q * _GPQ, (q + 1) * _GPQ, dot, 0)

  pltpu.sync_copy(out_v, out_hbm.at[pl.ds(base, _BPW)])


@jax.jit
def _run(xf, tflat):
  mesh = plsc.VectorSubcoreMesh(core_axis_name="c", subcore_axis_name="s",
                                num_cores=_NC, num_subcores=_NS)
  return pl.kernel(
      _body,
      out_type=jax.ShapeDtypeStruct((_B,), jnp.float32),
      mesh=mesh,
      compiler_params=pltpu.CompilerParams(needs_layout_passes=False,
                                           use_tc_tiling_on_sc=False),
      scratch_types=[
          pltpu.VMEM((_BPW,), jnp.int32),
          pltpu.VMEM((_BPW,), jnp.int32),
          pltpu.VMEM((_EPW,), jnp.int32),
          pltpu.VMEM((_EPW,), jnp.float32),
          pltpu.VMEM((_BPW,), jnp.float32),
          pltpu.SemaphoreType.DMA,
          pltpu.SemaphoreType.DMA,
          pltpu.SemaphoreType.DMA,
          pltpu.SemaphoreType.DMA,
      ],
  )(xf, tflat)


def kernel(x, table):
  n, b = table.shape[0], x.shape[0]
  xf = (jnp.asarray(x, jnp.int32).T.reshape(2, b // 128, 128)
        .transpose(1, 0, 2).reshape(2 * b))
  tflat = (table.T.reshape(2, 8, n // 128, 128)
           .transpose(0, 2, 1, 3).reshape(n * _D))
  return _run(xf, tflat).reshape(b, 1)
